# baseline (speedup 1.0000x reference)
import functools

import jax
import jax.numpy as jnp
from jax import lax
from jax.experimental import pallas as pl
from jax.experimental.pallas import tpu as pltpu
from jax.experimental.pallas import tpu_sc as plsc

BATCH = 16384
K_EMB = 32
NC = 2
NS = 16
NW = NC * NS
BPW = BATCH // NW
CHUNK = 128
NCH = BPW // CHUNK

BLK = 2048


def _sc_gather(user_table, item_table, u3, i3):
    mesh = plsc.VectorSubcoreMesh(
        core_axis_name="c", subcore_axis_name="s", num_cores=NC)

    @functools.partial(
        pl.kernel,
        mesh=mesh,
        compiler_params=pltpu.CompilerParams(use_tc_tiling_on_sc=False),
        out_type=(
            jax.ShapeDtypeStruct((BATCH, K_EMB), jnp.float32),
            jax.ShapeDtypeStruct((BATCH, K_EMB), jnp.float32),
        ),
        scratch_types=[
            pltpu.VMEM((NCH, CHUNK), jnp.int32),
            pltpu.VMEM((NCH, CHUNK), jnp.int32),
            pltpu.VMEM((BPW, K_EMB), jnp.float32),
            pltpu.VMEM((BPW, K_EMB), jnp.float32),
            pltpu.SemaphoreType.DMA,
            pltpu.SemaphoreType.DMA,
        ],
    )
    def gather_k(ut_hbm, it_hbm, u_hbm, i_hbm, ue_out, ie_out,
                 uidx_v, iidx_v, urows_v, irows_v, usem, isem):
        wid = lax.axis_index("s") * NC + lax.axis_index("c")
        base = wid * BPW
        pltpu.sync_copy(u_hbm.at[wid], uidx_v)
        pltpu.sync_copy(i_hbm.at[wid], iidx_v)
        ucps = []
        icps = []
        for j in range(NCH):
            ucps.append(pltpu.async_copy(
                ut_hbm.at[uidx_v.at[j]],
                urows_v.at[pl.ds(j * CHUNK, CHUNK)], usem))
            icps.append(pltpu.async_copy(
                it_hbm.at[iidx_v.at[j]],
                irows_v.at[pl.ds(j * CHUNK, CHUNK)], isem))
        for cp in ucps:
            cp.wait()
        for cp in icps:
            cp.wait()
        pltpu.sync_copy(urows_v, ue_out.at[pl.ds(base, BPW)])
        pltpu.sync_copy(irows_v, ie_out.at[pl.ds(base, BPW)])

    return gather_k(user_table, item_table, u3, i3)


def _mlp_body(ue_ref, ie_ref, gs_ref, w1u_ref, w1i_ref, w1gs_ref, b1_ref,
              w2_ref, b2_ref, w3_ref, b3_ref, out_ref):
    x1 = jnp.dot(ue_ref[:], w1u_ref[:], preferred_element_type=jnp.float32,
                 precision=lax.Precision.HIGHEST)
    x1 = x1 + jnp.dot(ie_ref[:], w1i_ref[:], preferred_element_type=jnp.float32,
                      precision=lax.Precision.HIGHEST)
    x1 = x1 + jnp.dot(gs_ref[:], w1gs_ref[:], preferred_element_type=jnp.float32,
                      precision=lax.Precision.HIGHEST)
    h1 = jnp.maximum(x1 + b1_ref[:], 0.0)
    h2 = jnp.dot(h1, w2_ref[:], preferred_element_type=jnp.float32,
                 precision=lax.Precision.HIGHEST)
    h2 = jnp.maximum(h2 + b2_ref[:], 0.0)
    out = jnp.dot(h2, w3_ref[:], preferred_element_type=jnp.float32,
                  precision=lax.Precision.HIGHEST)
    out_ref[:] = out + b3_ref[:]


def kernel(u, i, g, s, user_table, item_table, W1, b1, W2, b2, W3, b3):
    u3 = u.astype(jnp.int32).reshape(NW, NCH, CHUNK)
    i3 = i.astype(jnp.int32).reshape(NW, NCH, CHUNK)
    ue, ie = _sc_gather(user_table, item_table, u3, i3)

    gs = jnp.concatenate([g, s[:, None]], axis=1)
    w1u = W1[:K_EMB]
    w1i = W1[K_EMB:2 * K_EMB]
    w1gs = W1[2 * K_EMB:]
    b1r = b1[None, :]
    b2r = b2[None, :]
    b3r = b3[None, :]

    grid = (BATCH // BLK,)
    out2d = pl.pallas_call(
        _mlp_body,
        grid=grid,
        in_specs=[
            pl.BlockSpec((BLK, K_EMB), lambda b: (b, 0)),
            pl.BlockSpec((BLK, K_EMB), lambda b: (b, 0)),
            pl.BlockSpec((BLK, gs.shape[1]), lambda b: (b, 0)),
            pl.BlockSpec(w1u.shape, lambda b: (0, 0)),
            pl.BlockSpec(w1i.shape, lambda b: (0, 0)),
            pl.BlockSpec(w1gs.shape, lambda b: (0, 0)),
            pl.BlockSpec(b1r.shape, lambda b: (0, 0)),
            pl.BlockSpec(W2.shape, lambda b: (0, 0)),
            pl.BlockSpec(b2r.shape, lambda b: (0, 0)),
            pl.BlockSpec(W3.shape, lambda b: (0, 0)),
            pl.BlockSpec(b3r.shape, lambda b: (0, 0)),
        ],
        out_specs=pl.BlockSpec((BLK, 1), lambda b: (b, 0)),
        out_shape=jax.ShapeDtypeStruct((BATCH, 1), jnp.float32),
    )(ue, ie, gs, w1u, w1i, w1gs, b1r, W2, b2r, W3, b3r)
    return out2d[:, 0]
